# baseline (speedup 1.0000x reference)
import jax
import jax.numpy as jnp
from jax import lax
from jax.experimental import pallas as pl
from jax.experimental.pallas import tpu as pltpu

_NPAD = 5120
_B = 1024
_C = 1024
_NBLK = _NPAD // _B
_CONF = 0.25
_IOU = 0.45


def _nms_body(s_row_ref, s_col_ref, data_ref, out_ref,
              rank_ref, scol_ref, sup_ref, area_ref):
    f32 = jnp.float32
    bf16 = jnp.bfloat16

    s_raw = s_row_ref[...]
    conf_mask = s_raw >= _CONF
    n_active = jnp.sum(conf_mask.astype(f32)).astype(jnp.int32)
    kmax = (n_active + _B - 1) // _B

    rank_ref[...] = jnp.zeros((1, _NPAD), f32)
    ones_row = jnp.ones((1, _B), bf16)
    rdims = (((1,), (0,)), ((), ()))
    ltri = (lax.broadcasted_iota(jnp.int32, (_B, _B), 0)
            < lax.broadcasted_iota(jnp.int32, (_B, _B), 1))

    for k in range(_NBLK):
        sk = s_col_ref[pl.ds(k * _B, _B), :]
        for c in range(_NBLK):
            srow_c = s_row_ref[0:1, pl.ds(c * _B, _B)]
            if c == k:
                m = ((sk > srow_c) | ((sk == srow_c) & ltri)).astype(bf16)
            elif c < k:
                m = (sk > srow_c).astype(bf16)
            else:
                m = (sk >= srow_c).astype(bf16)
            rank_ref[0:1, pl.ds(c * _B, _B)] += lax.dot_general(
                ones_row, m, rdims, preferred_element_type=f32)

    data24 = data_ref[...]
    rank_row = rank_ref[...]
    dims = (((1,), (1,)), ((), ()))

    for r in range(_NBLK):
        @pl.when(r < kmax)
        def _gather(r=r):
            rloc = (lax.broadcasted_iota(jnp.int32, (_B, 1), 0)
                    + r * _B).astype(f32)
            oh = (rank_row == rloc).astype(bf16)
            g24 = lax.dot_general(oh, data24, dims,
                                  preferred_element_type=f32)
            g = g24[:, 0:8] + g24[:, 8:16] + g24[:, 16:24]
            scol_ref[pl.ds(r * _B, _B), :] = g
            out_ref[:, pl.ds(r * _B, _B)] = g.T

        @pl.when(r >= kmax)
        def _ztail(r=r):
            out_ref[:, pl.ds(r * _B, _B)] = jnp.zeros((8, _B), f32)

    sup_ref[...] = (out_ref[4:5, :] < _CONF).astype(f32)
    area_ref[...] = ((out_ref[2:3, :] - out_ref[0:1, :])
                     * (out_ref[3:4, :] - out_ref[1:2, :]))

    def p3(k, base):
        blk = scol_ref[pl.ds(base, _B), :]
        x1i = blk[:, 0:1]
        y1i = blk[:, 1:2]
        x2i = blk[:, 2:3]
        y2i = blk[:, 3:4]
        area_i = (x2i - x1i) * (y2i - y1i)

        x1r = out_ref[0:1, pl.ds(base, _B)]
        y1r = out_ref[1:2, pl.ds(base, _B)]
        x2r = out_ref[2:3, pl.ds(base, _B)]
        y2r = out_ref[3:4, pl.ds(base, _B)]
        area_r = (x2r - x1r) * (y2r - y1r)
        wl = jnp.clip(jnp.minimum(x2i, x2r) - jnp.maximum(x1i, x1r), 0.0)
        hl = jnp.clip(jnp.minimum(y2i, y2r) - jnp.maximum(y1i, y1r), 0.0)
        interl = wl * hl
        ioul = interl / (area_i + area_r - interl + 1e-9)
        tri = (lax.broadcasted_iota(jnp.int32, (_B, _B), 0)
               < lax.broadcasted_iota(jnp.int32, (_B, _B), 1))
        etri = ((ioul > _IOU) & tri).astype(bf16)

        base_sup = sup_ref[0:1, pl.ds(base, _B)]

        def fp_cond(c):
            return c[1]

        def fp_body(c):
            s, _ = c
            contrib = lax.dot_general((1.0 - s).astype(bf16), etri,
                                      (((1,), (0,)), ((), ())),
                                      preferred_element_type=f32)
            ns = jnp.maximum(base_sup, (contrib > 0.0).astype(f32))
            return ns, jnp.any(ns != s)

        suploc, _ = lax.while_loop(fp_cond, fp_body, (base_sup, True))
        sup_ref[0:1, pl.ds(base, _B)] = suploc

        keeprow = (1.0 - suploc).astype(bf16)

        def chunk(cb):
            x1a = out_ref[0:1, pl.ds(cb, _C)]
            y1a = out_ref[1:2, pl.ds(cb, _C)]
            x2a = out_ref[2:3, pl.ds(cb, _C)]
            y2a = out_ref[3:4, pl.ds(cb, _C)]
            area_a = area_ref[0:1, pl.ds(cb, _C)]
            w = jnp.clip(jnp.minimum(x2i, x2a) - jnp.maximum(x1i, x1a), 0.0)
            h = jnp.clip(jnp.minimum(y2i, y2a) - jnp.maximum(y1i, y1a), 0.0)
            inter = w * h
            iou = inter / (area_i + area_a - inter + 1e-9)
            e = (iou > _IOU).astype(bf16)
            contrib = lax.dot_general(keeprow, e, (((1,), (0,)), ((), ())),
                                      preferred_element_type=f32)
            supd = (contrib > 0.0).astype(f32)
            sup_ref[0:1, pl.ds(cb, _C)] = jnp.maximum(
                sup_ref[0:1, pl.ds(cb, _C)], supd)

        return chunk

    for k in range(_NBLK):
        @pl.when(k < kmax)
        def _blk(k=k):
            chunk = p3(k, k * _B)
            for c in range(k + 1, _NBLK):
                @pl.when(c < kmax)
                def _chk(c=c, chunk=chunk):
                    chunk(c * _C)

    keep = 1.0 - sup_ref[...]
    out_ref[...] = out_ref[...] * keep


@jax.jit
def kernel(boxes, scores):
    n = boxes.shape[0]
    pad = _NPAD - n
    s_pad = jnp.pad(scores.astype(jnp.float32), (0, pad), constant_values=-1.0)
    b_pad = jnp.pad(boxes.astype(jnp.float32), ((0, pad), (0, 0)))
    data = jnp.concatenate(
        [b_pad.T, s_pad[None, :], jnp.zeros((3, _NPAD), jnp.float32)], axis=0)
    def _trunc_bf16(x):
        t = lax.bitcast_convert_type(x, jnp.uint32) & jnp.uint32(0xFFFF0000)
        return lax.bitcast_convert_type(t, jnp.float32)

    hi_f = _trunc_bf16(data)
    r1 = data - hi_f
    mid_f = _trunc_bf16(r1)
    data24 = jnp.concatenate(
        [hi_f.astype(jnp.bfloat16),
         mid_f.astype(jnp.bfloat16),
         (r1 - mid_f).astype(jnp.bfloat16)], axis=0)

    out = pl.pallas_call(
        _nms_body,
        out_shape=jax.ShapeDtypeStruct((8, _NPAD), jnp.float32),
        scratch_shapes=[
            pltpu.VMEM((1, _NPAD), jnp.float32),
            pltpu.VMEM((_NPAD, 8), jnp.float32),
            pltpu.VMEM((1, _NPAD), jnp.float32),
            pltpu.VMEM((1, _NPAD), jnp.float32),
        ],
    )(s_pad[None, :], s_pad[:, None], data24)
    return out[:5, :n].T


# speedup vs baseline: 1.2807x; 1.2807x over previous
import jax
import jax.numpy as jnp
from jax import lax
from jax.experimental import pallas as pl
from jax.experimental.pallas import tpu as pltpu

_NPAD = 5120
_B = 1024
_C = 1024
_NBLK = _NPAD // _B
_CONF = 0.25
_IOU = 0.45


def _nms_body(s_row_ref, s_col_ref, data_ref, out_ref,
              rank_ref, scol_ref, sup_ref, area_ref):
    f32 = jnp.float32
    bf16 = jnp.bfloat16

    s_raw = s_row_ref[...]
    conf_mask = s_raw >= _CONF
    n_active = jnp.sum(conf_mask.astype(f32)).astype(jnp.int32)
    kmax = (n_active + _B - 1) // _B

    rank_ref[...] = jnp.zeros((1, _NPAD), f32)
    ones_row = jnp.ones((1, _B), bf16)
    rdims = (((1,), (0,)), ((), ()))
    ltri = (lax.broadcasted_iota(jnp.int32, (_B, _B), 0)
            < lax.broadcasted_iota(jnp.int32, (_B, _B), 1))

    for k in range(_NBLK):
        sk = s_col_ref[pl.ds(k * _B, _B), :]
        for c in range(_NBLK):
            srow_c = s_row_ref[0:1, pl.ds(c * _B, _B)]
            if c == k:
                m = ((sk > srow_c) | ((sk == srow_c) & ltri)).astype(bf16)
            elif c < k:
                m = (sk > srow_c).astype(bf16)
            else:
                m = (sk >= srow_c).astype(bf16)
            rank_ref[0:1, pl.ds(c * _B, _B)] += lax.dot_general(
                ones_row, m, rdims, preferred_element_type=f32)

    data24 = data_ref[...]
    rank_row = rank_ref[...]
    dims = (((1,), (1,)), ((), ()))

    def ztail(r, carry):
        out_ref[:, pl.ds(r * _B, _B)] = jnp.zeros((8, _B), f32)
        return carry

    lax.fori_loop(kmax, _NBLK, ztail, 0)

    def p2(r, carry):
        rloc = (lax.broadcasted_iota(jnp.int32, (_B, 1), 0) + r * _B).astype(f32)
        oh = (rank_row == rloc).astype(bf16)
        g24 = lax.dot_general(oh, data24, dims,
                              preferred_element_type=f32)
        g = g24[:, 0:8] + g24[:, 8:16] + g24[:, 16:24]
        scol_ref[pl.ds(r * _B, _B), :] = g
        out_ref[:, pl.ds(r * _B, _B)] = g.T
        return carry

    lax.fori_loop(0, kmax, p2, 0)

    sup_ref[...] = (out_ref[4:5, :] < _CONF).astype(f32)
    area_ref[...] = ((out_ref[2:3, :] - out_ref[0:1, :])
                     * (out_ref[3:4, :] - out_ref[1:2, :]))

    def p3(k, carry):
        base = k * _B
        blk = scol_ref[pl.ds(base, _B), :]
        x1i = blk[:, 0:1]
        y1i = blk[:, 1:2]
        x2i = blk[:, 2:3]
        y2i = blk[:, 3:4]
        area_i = (x2i - x1i) * (y2i - y1i)

        x1r = out_ref[0:1, pl.ds(base, _B)]
        y1r = out_ref[1:2, pl.ds(base, _B)]
        x2r = out_ref[2:3, pl.ds(base, _B)]
        y2r = out_ref[3:4, pl.ds(base, _B)]
        area_r = (x2r - x1r) * (y2r - y1r)
        wl = jnp.clip(jnp.minimum(x2i, x2r) - jnp.maximum(x1i, x1r), 0.0)
        hl = jnp.clip(jnp.minimum(y2i, y2r) - jnp.maximum(y1i, y1r), 0.0)
        interl = wl * hl
        ioul = interl / (area_i + area_r - interl + 1e-9)
        tri = (lax.broadcasted_iota(jnp.int32, (_B, _B), 0)
               < lax.broadcasted_iota(jnp.int32, (_B, _B), 1))
        etri = ((ioul > _IOU) & tri).astype(bf16)

        base_sup = sup_ref[0:1, pl.ds(base, _B)]

        def fp_cond(c):
            return c[1]

        def fp_body(c):
            s, _ = c
            contrib = lax.dot_general((1.0 - s).astype(bf16), etri,
                                      (((1,), (0,)), ((), ())),
                                      preferred_element_type=f32)
            ns = jnp.maximum(base_sup, (contrib > 0.0).astype(f32))
            return ns, jnp.any(ns != s)

        suploc, _ = lax.while_loop(fp_cond, fp_body, (base_sup, True))
        sup_ref[0:1, pl.ds(base, _B)] = suploc

        keeprow = (1.0 - suploc).astype(bf16)

        def chunk(c, carry2):
            cb = c * _C
            x1a = out_ref[0:1, pl.ds(cb, _C)]
            y1a = out_ref[1:2, pl.ds(cb, _C)]
            x2a = out_ref[2:3, pl.ds(cb, _C)]
            y2a = out_ref[3:4, pl.ds(cb, _C)]
            area_a = area_ref[0:1, pl.ds(cb, _C)]
            w = jnp.clip(jnp.minimum(x2i, x2a) - jnp.maximum(x1i, x1a), 0.0)
            h = jnp.clip(jnp.minimum(y2i, y2a) - jnp.maximum(y1i, y1a), 0.0)
            inter = w * h
            iou = inter / (area_i + area_a - inter + 1e-9)
            e = (iou > _IOU).astype(bf16)
            contrib = lax.dot_general(keeprow, e, (((1,), (0,)), ((), ())),
                                      preferred_element_type=f32)
            jc = lax.broadcasted_iota(jnp.int32, (1, _C), 1) + cb
            supd = ((contrib > 0.0) & (jc >= base + _B)).astype(f32)
            sup_ref[0:1, pl.ds(cb, _C)] = jnp.maximum(
                sup_ref[0:1, pl.ds(cb, _C)], supd)
            return carry2

        lax.fori_loop((base + _B) // _C, (kmax * _B + _C - 1) // _C, chunk, 0)
        return carry

    lax.fori_loop(0, kmax, p3, 0)

    keep = 1.0 - sup_ref[...]
    out_ref[...] = out_ref[...] * keep


@jax.jit
def kernel(boxes, scores):
    n = boxes.shape[0]
    pad = _NPAD - n
    s_pad = jnp.pad(scores.astype(jnp.float32), (0, pad), constant_values=-1.0)
    b_pad = jnp.pad(boxes.astype(jnp.float32), ((0, pad), (0, 0)))
    data = jnp.concatenate(
        [b_pad.T, s_pad[None, :], jnp.zeros((3, _NPAD), jnp.float32)], axis=0)
    def _trunc_bf16(x):
        t = lax.bitcast_convert_type(x, jnp.uint32) & jnp.uint32(0xFFFF0000)
        return lax.bitcast_convert_type(t, jnp.float32)

    hi_f = _trunc_bf16(data)
    r1 = data - hi_f
    mid_f = _trunc_bf16(r1)
    data24 = jnp.concatenate(
        [hi_f.astype(jnp.bfloat16),
         mid_f.astype(jnp.bfloat16),
         (r1 - mid_f).astype(jnp.bfloat16)], axis=0)

    out = pl.pallas_call(
        _nms_body,
        out_shape=jax.ShapeDtypeStruct((8, _NPAD), jnp.float32),
        scratch_shapes=[
            pltpu.VMEM((1, _NPAD), jnp.float32),
            pltpu.VMEM((_NPAD, 8), jnp.float32),
            pltpu.VMEM((1, _NPAD), jnp.float32),
            pltpu.VMEM((1, _NPAD), jnp.float32),
        ],
    )(s_pad[None, :], s_pad[:, None], data24)
    return out[:5, :n].T
